# baseline (speedup 1.0000x reference)
import jax
import jax.numpy as jnp
from jax.experimental import pallas as pl
from jax.experimental.pallas import tpu as pltpu

MS = 512
TRIU = MS * (MS + 1) // 2
ROWS128 = TRIU // 128
WIN = 24


def _body(in_ref, out_ref):
    lane = jax.lax.broadcasted_iota(jnp.int32, (8, 128), 1)
    j128 = jax.lax.broadcasted_iota(jnp.int32, (4, 128), 0) * 128
    col = j128 + lane[:4]

    def row(r, _):
        s = 512 * r - (r * (r + 1)) // 2
        q = s // 128
        m = s - q * 128
        q8 = pl.multiple_of(jnp.minimum(8 * (q // 8), ROWS128 - WIN), 8)
        d = q - q8
        t = in_ref[0, pl.ds(q8, WIN), :]
        t = pltpu.roll(t, WIN - d, axis=0)
        r1 = pltpu.roll(t[0:8], 128 - m, axis=1)
        r2 = pltpu.roll(t[1:9], 128 - m, axis=1)
        vals = jnp.where(lane < 128 - m, r1, r2)[:4]
        out_ref[0, r] = jnp.where(col >= r, vals, 0.0)
        return 0

    jax.lax.fori_loop(0, MS, row, 0)


def kernel(inputs):
    batch = inputs.shape[0]
    out = pl.pallas_call(
        _body,
        grid=(batch,),
        in_specs=[pl.BlockSpec((1, ROWS128, 128), lambda b: (b, 0, 0))],
        out_specs=pl.BlockSpec((1, MS, 4, 128), lambda b: (b, 0, 0, 0)),
        out_shape=jax.ShapeDtypeStruct((batch, MS, 4, 128), inputs.dtype),
    )(inputs.reshape(batch, ROWS128, 128))
    return out.reshape(batch, MS, MS)


# speedup vs baseline: 10.7287x; 10.7287x over previous
import functools

import jax
import jax.numpy as jnp
from jax import lax
from jax.experimental import pallas as pl
from jax.experimental.pallas import tpu as pltpu
from jax.experimental.pallas import tpu_sc as plsc

MS = 512
TRIU = MS * (MS + 1) // 2
BATCH = 128
RB = 32
NBLK = MS // RB
NW = 32
SPB = BATCH // NW


def _s(r):
    return 512 * r - (r * (r + 1)) // 2


_STARTS = []
_LENS = []
for _i in range(NBLK):
    _r0 = RB * _i
    _st = (_s(_r0) // 8) * 8
    _end = min(_s(_r0 + RB - 1) + MS, TRIU)
    _ln = -((-(_end - _st)) // 8) * 8
    _STARTS.append(_st)
    _LENS.append(_ln)
MAXSPAN = max(_LENS)


def _sc_body(in_hbm, out_hbm, in_v, out_v):
    wid = lax.axis_index("s") * 2 + lax.axis_index("c")

    def sample_body(t, _):
        b = wid * SPB + t
        for blk in range(NBLK):
            r0 = RB * blk
            start = _STARTS[blk]
            ln = _LENS[blk]
            off = pl.multiple_of(b * TRIU + start, 8)
            pltpu.sync_copy(in_hbm.at[pl.ds(off, ln)],
                            in_v.at[pl.ds(0, ln)])

            def row_body(r, _):
                sr = 512 * r - (r * (r + 1)) // 2 - start
                rloc = r - r0
                m = r // 16

                def zero_body(c, _):
                    out_v[rloc, pl.ds(c * 16, 16)] = jnp.zeros((16,), jnp.float32)
                    return 0

                lax.fori_loop(0, m, zero_body, 0)

                v = in_v[pl.ds(sr + m * 16, 16)]
                col = lax.iota(jnp.int32, 16) + m * 16
                out_v[rloc, pl.ds(m * 16, 16)] = jnp.where(col >= r, v, 0.0)

                def copy_body(c, _):
                    out_v[rloc, pl.ds(c * 16, 16)] = in_v[pl.ds(sr + c * 16, 16)]
                    return 0

                lax.fori_loop(m + 1, 32, copy_body, 0)
                return 0

            lax.fori_loop(r0, r0 + RB, row_body, 0)
            pltpu.sync_copy(out_v, out_hbm.at[b, pl.ds(r0, RB), :])
        return 0

    lax.fori_loop(0, SPB, sample_body, 0)


def kernel(inputs):
    mesh = plsc.VectorSubcoreMesh(core_axis_name="c", subcore_axis_name="s")
    run = functools.partial(
        pl.kernel,
        mesh=mesh,
        out_type=jax.ShapeDtypeStruct((BATCH, MS, MS), jnp.float32),
        scratch_types=[
            pltpu.VMEM((MAXSPAN,), jnp.float32),
            pltpu.VMEM((RB, MS), jnp.float32),
        ],
    )

    @run
    def _k(in_hbm, out_hbm, in_v, out_v):
        _sc_body(in_hbm, out_hbm, in_v, out_v)

    return _k(inputs.reshape(-1))


# speedup vs baseline: 26.0213x; 2.4254x over previous
import functools

import jax
import jax.numpy as jnp
from jax import lax
from jax.experimental import pallas as pl
from jax.experimental.pallas import tpu as pltpu
from jax.experimental.pallas import tpu_sc as plsc

MS = 512
TRIU = MS * (MS + 1) // 2
BATCH = 128
RB = 32
NBLK = MS // RB
NCH = MS // 16
NW = 32
SPB = BATCH // NW


def _s(r):
    return 512 * r - (r * (r + 1)) // 2


_STARTS = []
_LENS = []
for _i in range(NBLK):
    _r0 = RB * _i
    _st = (_s(_r0) // 8) * 8
    _end = min(_s(_r0 + RB - 1) + MS, TRIU)
    _ln = -((-(_end - _st)) // 8) * 8
    _STARTS.append(_st)
    _LENS.append(_ln)
MAXSPAN = max(_LENS)

def _issue_in(in_hbm, iv, sem, b, blk):
    off = pl.multiple_of(b * TRIU + _STARTS[blk], 8)
    pltpu.async_copy(in_hbm.at[pl.ds(off, _LENS[blk])],
                     iv.at[pl.ds(0, _LENS[blk])], sem)


def _wait_in(in_hbm, iv, sem, blk):
    pltpu.make_async_copy(in_hbm.at[pl.ds(0, _LENS[blk])],
                          iv.at[pl.ds(0, _LENS[blk])], sem).wait()


def _compute_block(iv, ov, blk):
    r0 = RB * blk
    start = _STARTS[blk]

    def row_body(r, _):
        sr = 512 * r - (r * (r + 1)) // 2 - start
        rloc = r - r0
        m = r // 16
        for i in range(4):
            ci = jnp.maximum(m - 4 + i, 0)
            ov[rloc, pl.ds(ci * 16, 16)] = jnp.zeros((16,), jnp.float32)
        v = iv[pl.ds(sr + m * 16, 16)]
        col = lax.iota(jnp.int32, 16) + m * 16
        ov[rloc, pl.ds(m * 16, 16)] = jnp.where(col >= r, v, 0.0)

        @plsc.parallel_loop(m + 1, NCH, unroll=4)
        def _copy(c):
            ov[rloc, pl.ds(c * 16, 16)] = iv[pl.ds(sr + c * 16, 16)]

        return 0

    lax.fori_loop(r0, r0 + RB, row_body, 0)


def _sc_body(in_hbm, out_hbm, iv0, iv1, ov0, ov1, si0, si1, so0, so1):
    wid = lax.axis_index("s") * 2 + lax.axis_index("c")
    ivs, ovs, sis, sos = (iv0, iv1), (ov0, ov1), (si0, si1), (so0, so1)

    for ov in (ov0, ov1):
        @plsc.parallel_loop(0, RB * NCH, unroll=4)
        def _zero(i):
            ov[i // NCH, pl.ds((i % NCH) * 16, 16)] = jnp.zeros((16,), jnp.float32)

    _issue_in(in_hbm, iv0, si0, wid * SPB, 0)

    def sample_body(t, _):
        b = wid * SPB + t
        for blk in range(NBLK):
            p = blk % 2
            _wait_in(in_hbm, ivs[p], sis[p], blk)
            if blk < NBLK - 1:
                _issue_in(in_hbm, ivs[1 - p], sis[1 - p], b, blk + 1)
            else:
                nb = jnp.minimum(b + 1, BATCH - 1)
                _issue_in(in_hbm, ivs[1 - p], sis[1 - p], nb, 0)

            wait_out = pltpu.make_async_copy(
                ovs[p], out_hbm.at[0, pl.ds(0, RB), :], sos[p])
            if blk >= 2:
                wait_out.wait()
            else:
                @pl.when(t > 0)
                def _():
                    wait_out.wait()

            _compute_block(ivs[p], ovs[p], blk)
            pltpu.async_copy(ovs[p], out_hbm.at[b, pl.ds(RB * blk, RB), :],
                             sos[p])
        return 0

    lax.fori_loop(0, SPB, sample_body, 0)

    for p in (0, 1):
        pltpu.make_async_copy(ovs[p], out_hbm.at[0, pl.ds(0, RB), :],
                              sos[p]).wait()
    _wait_in(in_hbm, iv0, si0, 0)


def kernel(inputs):
    mesh = plsc.VectorSubcoreMesh(core_axis_name="c", subcore_axis_name="s")
    run = functools.partial(
        pl.kernel,
        mesh=mesh,
        out_type=jax.ShapeDtypeStruct((BATCH, MS, MS), jnp.float32),
        scratch_types=[
            pltpu.VMEM((MAXSPAN,), jnp.float32),
            pltpu.VMEM((MAXSPAN,), jnp.float32),
            pltpu.VMEM((RB, MS), jnp.float32),
            pltpu.VMEM((RB, MS), jnp.float32),
            pltpu.SemaphoreType.DMA,
            pltpu.SemaphoreType.DMA,
            pltpu.SemaphoreType.DMA,
            pltpu.SemaphoreType.DMA,
        ],
    )

    @run
    def _k(in_hbm, out_hbm, iv0, iv1, ov0, ov1, si0, si1, so0, so1):
        _sc_body(in_hbm, out_hbm, iv0, iv1, ov0, ov1, si0, si1, so0, so1)

    return _k(inputs.reshape(-1))
